# Initial kernel scaffold; baseline (speedup 1.0000x reference)
import jax
import jax.numpy as jnp
from jax.experimental import pallas as pl


def kernel(x, edge_index, edge_attr, batch, Wf, bf, Ws, bs, Wg, bg, W3, b3, gamma, beta, W4, b4):
    raise NotImplementedError("write your pallas kernel here")


# speedup vs baseline: 158.7672x; 158.7672x over previous
import functools

import jax
import jax.numpy as jnp
import numpy as np
from jax import lax
from jax.experimental import pallas as pl
from jax.experimental.pallas import tpu as pltpu
from jax.experimental.pallas import tpu_sc as plsc

N = 100000
E = 3200000
B = 100
NODE_ATOM = 1000
LANES = 16

ROWS = E // 128
CH_ROWS = 8
NCHUNK = ROWS // CH_ROWS
NWORK = 32
GMAX = (NCHUNK + NWORK - 1) // NWORK

SL_FULL = 6272
SL_LAST = N - 15 * SL_FULL

_mesh = plsc.VectorSubcoreMesh(core_axis_name="c", subcore_axis_name="s")


def _zero_spmem_slices(s, zbuf, shared_refs):
    def zb(i, t):
        zbuf[pl.ds(i * LANES, LANES)] = jnp.zeros((LANES,), jnp.float32)
        return t

    lax.fori_loop(0, SL_FULL // LANES, zb, 0)

    @pl.when(s < 15)
    def _():
        for ref in shared_refs:
            pltpu.sync_copy(zbuf.at[pl.ds(0, SL_FULL)],
                            ref.at[pl.ds(s * SL_FULL, SL_FULL)])

    @pl.when(s == 15)
    def _():
        for ref in shared_refs:
            pltpu.sync_copy(zbuf.at[pl.ds(0, SL_LAST)],
                            ref.at[pl.ds(15 * SL_FULL, SL_LAST)])


def _write_out_slices(s, zbuf, pairs):
    @pl.when(s < 15)
    def _():
        for ref, out in pairs:
            pltpu.sync_copy(ref.at[pl.ds(s * SL_FULL, SL_FULL)],
                            zbuf.at[pl.ds(0, SL_FULL)])
            pltpu.sync_copy(zbuf.at[pl.ds(0, SL_FULL)],
                            out.at[pl.ds(s * SL_FULL, SL_FULL)])

    @pl.when(s == 15)
    def _():
        for ref, out in pairs:
            pltpu.sync_copy(ref.at[pl.ds(15 * SL_FULL, SL_LAST)],
                            zbuf.at[pl.ds(0, SL_LAST)])
            pltpu.sync_copy(zbuf.at[pl.ds(0, SL_LAST)],
                            out.at[pl.ds(15 * SL_FULL, SL_LAST)])


@functools.partial(
    pl.kernel,
    out_type=[jax.ShapeDtypeStruct((N,), jnp.float32),
              jax.ShapeDtypeStruct((N,), jnp.float32),
              jax.ShapeDtypeStruct((N,), jnp.float32),
              jax.ShapeDtypeStruct((N,), jnp.float32)],
    mesh=_mesh,
    compiler_params=pltpu.CompilerParams(needs_layout_passes=False),
    scratch_types=[
        pltpu.VMEM((N,), jnp.float32),
        pltpu.VMEM((CH_ROWS, 128), jnp.int32),
        pltpu.VMEM((CH_ROWS, 128), jnp.int32),
        pltpu.VMEM((CH_ROWS, 128), jnp.float32),
        pltpu.VMEM((CH_ROWS, 128), jnp.float32),
        pltpu.VMEM((8, LANES), jnp.float32),
        pltpu.VMEM((SL_FULL,), jnp.float32),
        pltpu.VMEM_SHARED((N,), jnp.float32),
        pltpu.VMEM_SHARED((N,), jnp.float32),
    ],
)
def _pass1(x_hbm, src_hbm, dst_hbm, ea_hbm, w_hbm,
           macc0_out, macc1_out, dacc0_out, dacc1_out,
           x_v, src_v, dst_v, ea_v, m_v, w_v, zbuf, macc_s, dacc_s):
    c = lax.axis_index("c")
    s = lax.axis_index("s")
    wid = s * 2 + c

    pltpu.sync_copy(x_hbm, x_v)
    pltpu.sync_copy(w_hbm, w_v)
    _zero_spmem_slices(s, zbuf, (macc_s, dacc_s))
    plsc.subcore_barrier()

    wf0 = w_v[0]
    wf1 = w_v[1]
    wf2 = w_v[2]
    bf0 = w_v[3]
    ws0 = w_v[4]
    ws1 = w_v[5]
    ws2 = w_v[6]
    bs0 = w_v[7]

    def chunk_body(g, tok):
        t = g * NWORK + wid

        @pl.when(t < NCHUNK)
        def _():
            r0 = t * CH_ROWS
            pltpu.sync_copy(src_hbm.at[pl.ds(r0, CH_ROWS)], src_v)
            pltpu.sync_copy(dst_hbm.at[pl.ds(r0, CH_ROWS)], dst_v)
            pltpu.sync_copy(ea_hbm.at[pl.ds(r0, CH_ROWS)], ea_v)

            def row_body(j, tok2):
                for k in range(128 // LANES):
                    sl = pl.ds(k * LANES, LANES)
                    sv = src_v[j, sl]
                    dv = dst_v[j, sl]
                    ev = ea_v[j, sl]
                    xs = plsc.load_gather(x_v, [sv])
                    xd = plsc.load_gather(x_v, [dv])
                    a = wf0 * xd + wf1 * xs + wf2 * ev + bf0
                    b = ws0 * xd + ws1 * xs + ws2 * ev + bs0
                    sig = 1.0 / (1.0 + jnp.exp(-a))
                    u = jnp.exp(-jnp.abs(b))
                    st = u / (u + 2.0)
                    s2 = st * st
                    poly = 1.0 + s2 * (
                        0.3333333333 + s2 * (0.2 + s2 * (
                            0.1428571429 + s2 * 0.1111111111)))
                    sp = jnp.maximum(b, 0.0) + 2.0 * st * poly
                    m_v[j, sl] = sig * sp
                return tok2

            lax.fori_loop(0, CH_ROWS, row_body, 0)

            def scat_body(j, tok2):
                pltpu.sync_copy(m_v.at[j], macc_s.at[dst_v.at[j]], add=True)
                pltpu.sync_copy(ea_v.at[j], dacc_s.at[dst_v.at[j]], add=True)
                return tok2

            lax.fori_loop(0, CH_ROWS, scat_body, 0)

        return tok

    lax.fori_loop(0, GMAX, chunk_body, 0)
    plsc.subcore_barrier()

    @pl.when(c == 0)
    def _():
        _write_out_slices(s, zbuf, ((macc_s, macc0_out), (dacc_s, dacc0_out)))

    @pl.when(c == 1)
    def _():
        _write_out_slices(s, zbuf, ((macc_s, macc1_out), (dacc_s, dacc1_out)))


@functools.partial(
    pl.kernel,
    out_type=[jax.ShapeDtypeStruct((N,), jnp.float32),
              jax.ShapeDtypeStruct((N,), jnp.float32)],
    mesh=_mesh,
    compiler_params=pltpu.CompilerParams(needs_layout_passes=False),
    scratch_types=[
        pltpu.VMEM((N,), jnp.float32),
        pltpu.VMEM((CH_ROWS, 128), jnp.int32),
        pltpu.VMEM((CH_ROWS, 128), jnp.int32),
        pltpu.VMEM((CH_ROWS, 128), jnp.float32),
        pltpu.VMEM((CH_ROWS, 128), jnp.float32),
        pltpu.VMEM((SL_FULL,), jnp.float32),
        pltpu.VMEM_SHARED((N,), jnp.float32),
    ],
)
def _pass2(p_hbm, src_hbm, dst_hbm, ea_hbm, qacc0_out, qacc1_out,
           p_v, src_v, dst_v, ea_v, m_v, zbuf, qacc_s):
    c = lax.axis_index("c")
    s = lax.axis_index("s")
    wid = s * 2 + c

    pltpu.sync_copy(p_hbm, p_v)
    _zero_spmem_slices(s, zbuf, (qacc_s,))
    plsc.subcore_barrier()

    def chunk_body(g, tok):
        t = g * NWORK + wid

        @pl.when(t < NCHUNK)
        def _():
            r0 = t * CH_ROWS
            pltpu.sync_copy(src_hbm.at[pl.ds(r0, CH_ROWS)], src_v)
            pltpu.sync_copy(dst_hbm.at[pl.ds(r0, CH_ROWS)], dst_v)
            pltpu.sync_copy(ea_hbm.at[pl.ds(r0, CH_ROWS)], ea_v)

            def row_body(j, tok2):
                for k in range(128 // LANES):
                    sl = pl.ds(k * LANES, LANES)
                    sv = src_v[j, sl]
                    ev = ea_v[j, sl]
                    ps = plsc.load_gather(p_v, [sv])
                    m_v[j, sl] = ev * ps
                return tok2

            lax.fori_loop(0, CH_ROWS, row_body, 0)

            def scat_body(j, tok2):
                pltpu.sync_copy(m_v.at[j], qacc_s.at[dst_v.at[j]], add=True)
                return tok2

            lax.fori_loop(0, CH_ROWS, scat_body, 0)

        return tok

    lax.fori_loop(0, GMAX, chunk_body, 0)
    plsc.subcore_barrier()

    @pl.when(c == 0)
    def _():
        _write_out_slices(s, zbuf, ((qacc_s, qacc0_out),))

    @pl.when(c == 1)
    def _():
        _write_out_slices(s, zbuf, ((qacc_s, qacc1_out),))


def _combine_body(x_ref, macc0_ref, macc1_ref, dacc0_ref, dacc1_ref,
                  wg_ref, p_ref, dis_ref):
    m = macc0_ref[...] + macc1_ref[...]
    x1 = jnp.maximum(x_ref[...] + m, 0.0)
    deg = dacc0_ref[...] + dacc1_ref[...]
    pos = deg > 0
    dis = jnp.where(pos, lax.rsqrt(jnp.where(pos, deg, 1.0)), 0.0)
    p_ref[...] = dis * x1 * wg_ref[0, 0]
    dis_ref[...] = dis


_combine = pl.pallas_call(
    _combine_body,
    out_shape=[jax.ShapeDtypeStruct((B, NODE_ATOM), jnp.float32),
               jax.ShapeDtypeStruct((B, NODE_ATOM), jnp.float32)],
)


def _head_body(q0_ref, q1_ref, dis_ref, bg_ref, w3_ref, b3_ref, g_ref, be_ref,
               w4_ref, b4_ref, out_ref):
    q = q0_ref[...] + q1_ref[...]
    xd = jnp.maximum(dis_ref[...] * q + bg_ref[0, 0], 0.0)
    h = lax.dot_general(xd, w3_ref[...], (((1,), (1,)), ((), ())),
                        preferred_element_type=jnp.float32)
    h = h + b3_ref[...]
    h = h * np.float32(1.0 / np.sqrt(1.0 + 1e-5)) * g_ref[...] + be_ref[...]
    h = jnp.maximum(h, 0.0)
    o = lax.dot_general(h, w4_ref[...], (((1,), (1,)), ((), ())),
                        preferred_element_type=jnp.float32)
    out_ref[...] = jnp.maximum(o + b4_ref[...], 0.0)


_head = pl.pallas_call(
    _head_body,
    out_shape=jax.ShapeDtypeStruct((B, 128), jnp.float32),
)


def kernel(x, edge_index, edge_attr, batch, Wf, bf, Ws, bs, Wg, bg,
           W3, b3, gamma, beta, W4, b4):
    xf = x[:, 0]
    src = edge_index[0].reshape(ROWS, 128)
    dst = edge_index[1].reshape(ROWS, 128)
    ea = edge_attr[:, 0].reshape(ROWS, 128)
    w8 = jnp.concatenate([Wf[:, 0], bf, Ws[:, 0], bs])
    w8 = jnp.broadcast_to(w8[:, None], (8, LANES))

    macc0, macc1, dacc0, dacc1 = _pass1(xf, src, dst, ea, w8)
    p2, dis2 = _combine(x.reshape(B, NODE_ATOM),
                        macc0.reshape(B, NODE_ATOM),
                        macc1.reshape(B, NODE_ATOM),
                        dacc0.reshape(B, NODE_ATOM),
                        dacc1.reshape(B, NODE_ATOM),
                        Wg.reshape(1, 1))
    q0, q1 = _pass2(p2.reshape(N), src, dst, ea)
    out = _head(q0.reshape(B, NODE_ATOM), q1.reshape(B, NODE_ATOM),
                dis2, bg.reshape(1, 1),
                W3, b3.reshape(1, -1), gamma.reshape(1, -1),
                beta.reshape(1, -1), W4, b4.reshape(1, -1))
    return out


# speedup vs baseline: 236.7393x; 1.4911x over previous
import functools

import jax
import jax.numpy as jnp
import numpy as np
from jax import lax
from jax.experimental import pallas as pl
from jax.experimental.pallas import tpu as pltpu
from jax.experimental.pallas import tpu_sc as plsc

N = 100000
E = 3200000
B = 100
NODE_ATOM = 1000
LANES = 16

ROWS = E // 128
CH_ROWS = 8
NCHUNK = ROWS // CH_ROWS
NWORK = 32
GMAX = (NCHUNK + NWORK - 1) // NWORK

SL_FULL = 6272
SL_LAST = N - 15 * SL_FULL

_mesh = plsc.VectorSubcoreMesh(core_axis_name="c", subcore_axis_name="s")


def _zero_spmem_slices(s, zbuf, shared_refs):
    def zb(i, t):
        zbuf[pl.ds(i * LANES, LANES)] = jnp.zeros((LANES,), jnp.float32)
        return t

    lax.fori_loop(0, SL_FULL // LANES, zb, 0)

    @pl.when(s < 15)
    def _():
        for ref in shared_refs:
            pltpu.sync_copy(zbuf.at[pl.ds(0, SL_FULL)],
                            ref.at[pl.ds(s * SL_FULL, SL_FULL)])

    @pl.when(s == 15)
    def _():
        for ref in shared_refs:
            pltpu.sync_copy(zbuf.at[pl.ds(0, SL_LAST)],
                            ref.at[pl.ds(15 * SL_FULL, SL_LAST)])


def _write_out_slices(s, zbuf, pairs):
    @pl.when(s < 15)
    def _():
        for ref, out in pairs:
            pltpu.sync_copy(ref.at[pl.ds(s * SL_FULL, SL_FULL)],
                            zbuf.at[pl.ds(0, SL_FULL)])
            pltpu.sync_copy(zbuf.at[pl.ds(0, SL_FULL)],
                            out.at[pl.ds(s * SL_FULL, SL_FULL)])

    @pl.when(s == 15)
    def _():
        for ref, out in pairs:
            pltpu.sync_copy(ref.at[pl.ds(15 * SL_FULL, SL_LAST)],
                            zbuf.at[pl.ds(0, SL_LAST)])
            pltpu.sync_copy(zbuf.at[pl.ds(0, SL_LAST)],
                            out.at[pl.ds(15 * SL_FULL, SL_LAST)])


@functools.partial(
    pl.kernel,
    out_type=[jax.ShapeDtypeStruct((N,), jnp.float32),
              jax.ShapeDtypeStruct((N,), jnp.float32),
              jax.ShapeDtypeStruct((N,), jnp.float32),
              jax.ShapeDtypeStruct((N,), jnp.float32)],
    mesh=_mesh,
    compiler_params=pltpu.CompilerParams(needs_layout_passes=False),
    scratch_types=[
        pltpu.VMEM((N,), jnp.float32),
        pltpu.VMEM((CH_ROWS, 128), jnp.int32),
        pltpu.VMEM((CH_ROWS, 128), jnp.int32),
        pltpu.VMEM((CH_ROWS, 128), jnp.float32),
        pltpu.VMEM((CH_ROWS, 128), jnp.float32),
        pltpu.VMEM((8, LANES), jnp.float32),
        pltpu.VMEM((SL_FULL,), jnp.float32),
        pltpu.VMEM_SHARED((N,), jnp.float32),
        pltpu.VMEM_SHARED((N,), jnp.float32),
        pltpu.SemaphoreType.DMA,
        pltpu.SemaphoreType.DMA,
    ],
)
def _pass1(x_hbm, src_hbm, dst_hbm, ea_hbm, w_hbm,
           macc0_out, macc1_out, dacc0_out, dacc1_out,
           x_v, src_v, dst_v, ea_v, m_v, w_v, zbuf, macc_s, dacc_s,
           in_sem, scat_sem):
    c = lax.axis_index("c")
    s = lax.axis_index("s")
    wid = s * 2 + c

    pltpu.sync_copy(x_hbm, x_v)
    pltpu.sync_copy(w_hbm, w_v)
    _zero_spmem_slices(s, zbuf, (macc_s, dacc_s))
    plsc.subcore_barrier()

    wf0 = w_v[0]
    wf1 = w_v[1]
    wf2 = w_v[2]
    bf0 = w_v[3]
    ws0 = w_v[4]
    ws1 = w_v[5]
    ws2 = w_v[6]
    bs0 = w_v[7]

    def chunk_body(g, tok):
        t = g * NWORK + wid

        @pl.when(t < NCHUNK)
        def _():
            r0 = t * CH_ROWS
            d1 = pltpu.async_copy(src_hbm.at[pl.ds(r0, CH_ROWS)], src_v,
                                  in_sem)
            d2 = pltpu.async_copy(dst_hbm.at[pl.ds(r0, CH_ROWS)], dst_v,
                                  in_sem)
            d3 = pltpu.async_copy(ea_hbm.at[pl.ds(r0, CH_ROWS)], ea_v,
                                  in_sem)
            d1.wait()
            d2.wait()
            d3.wait()

            def row_body(j, tok2):
                for k in range(128 // LANES):
                    sl = pl.ds(k * LANES, LANES)
                    sv = src_v[j, sl]
                    dv = dst_v[j, sl]
                    ev = ea_v[j, sl]
                    xs = plsc.load_gather(x_v, [sv])
                    xd = plsc.load_gather(x_v, [dv])
                    a = wf0 * xd + wf1 * xs + wf2 * ev + bf0
                    b = ws0 * xd + ws1 * xs + ws2 * ev + bs0
                    sig = 1.0 / (1.0 + jnp.exp(-a))
                    u = jnp.exp(-jnp.abs(b))
                    st = u / (u + 2.0)
                    s2 = st * st
                    poly = 1.0 + s2 * (
                        0.3333333333 + s2 * (0.2 + s2 * (
                            0.1428571429 + s2 * 0.1111111111)))
                    sp = jnp.maximum(b, 0.0) + 2.0 * st * poly
                    m_v[j, sl] = sig * sp
                return tok2

            lax.fori_loop(0, CH_ROWS, row_body, 0)

            descs = []
            for j in range(CH_ROWS):
                descs.append(pltpu.async_copy(
                    m_v.at[j], macc_s.at[dst_v.at[j]], scat_sem, add=True))
                descs.append(pltpu.async_copy(
                    ea_v.at[j], dacc_s.at[dst_v.at[j]], scat_sem, add=True))
            for d in descs:
                d.wait()

        return tok

    lax.fori_loop(0, GMAX, chunk_body, 0)
    plsc.subcore_barrier()

    @pl.when(c == 0)
    def _():
        _write_out_slices(s, zbuf, ((macc_s, macc0_out), (dacc_s, dacc0_out)))

    @pl.when(c == 1)
    def _():
        _write_out_slices(s, zbuf, ((macc_s, macc1_out), (dacc_s, dacc1_out)))


@functools.partial(
    pl.kernel,
    out_type=[jax.ShapeDtypeStruct((N,), jnp.float32),
              jax.ShapeDtypeStruct((N,), jnp.float32)],
    mesh=_mesh,
    compiler_params=pltpu.CompilerParams(needs_layout_passes=False),
    scratch_types=[
        pltpu.VMEM((N,), jnp.float32),
        pltpu.VMEM((CH_ROWS, 128), jnp.int32),
        pltpu.VMEM((CH_ROWS, 128), jnp.int32),
        pltpu.VMEM((CH_ROWS, 128), jnp.float32),
        pltpu.VMEM((CH_ROWS, 128), jnp.float32),
        pltpu.VMEM((SL_FULL,), jnp.float32),
        pltpu.VMEM_SHARED((N,), jnp.float32),
        pltpu.SemaphoreType.DMA,
        pltpu.SemaphoreType.DMA,
    ],
)
def _pass2(p_hbm, src_hbm, dst_hbm, ea_hbm, qacc0_out, qacc1_out,
           p_v, src_v, dst_v, ea_v, m_v, zbuf, qacc_s, in_sem, scat_sem):
    c = lax.axis_index("c")
    s = lax.axis_index("s")
    wid = s * 2 + c

    pltpu.sync_copy(p_hbm, p_v)
    _zero_spmem_slices(s, zbuf, (qacc_s,))
    plsc.subcore_barrier()

    def chunk_body(g, tok):
        t = g * NWORK + wid

        @pl.when(t < NCHUNK)
        def _():
            r0 = t * CH_ROWS
            d1 = pltpu.async_copy(src_hbm.at[pl.ds(r0, CH_ROWS)], src_v,
                                  in_sem)
            d2 = pltpu.async_copy(dst_hbm.at[pl.ds(r0, CH_ROWS)], dst_v,
                                  in_sem)
            d3 = pltpu.async_copy(ea_hbm.at[pl.ds(r0, CH_ROWS)], ea_v,
                                  in_sem)
            d1.wait()
            d2.wait()
            d3.wait()

            def row_body(j, tok2):
                for k in range(128 // LANES):
                    sl = pl.ds(k * LANES, LANES)
                    sv = src_v[j, sl]
                    ev = ea_v[j, sl]
                    ps = plsc.load_gather(p_v, [sv])
                    m_v[j, sl] = ev * ps
                return tok2

            lax.fori_loop(0, CH_ROWS, row_body, 0)

            descs = []
            for j in range(CH_ROWS):
                descs.append(pltpu.async_copy(
                    m_v.at[j], qacc_s.at[dst_v.at[j]], scat_sem, add=True))
            for d in descs:
                d.wait()

        return tok

    lax.fori_loop(0, GMAX, chunk_body, 0)
    plsc.subcore_barrier()

    @pl.when(c == 0)
    def _():
        _write_out_slices(s, zbuf, ((qacc_s, qacc0_out),))

    @pl.when(c == 1)
    def _():
        _write_out_slices(s, zbuf, ((qacc_s, qacc1_out),))


def _combine_body(x_ref, macc0_ref, macc1_ref, dacc0_ref, dacc1_ref,
                  wg_ref, p_ref, dis_ref):
    m = macc0_ref[...] + macc1_ref[...]
    x1 = jnp.maximum(x_ref[...] + m, 0.0)
    deg = dacc0_ref[...] + dacc1_ref[...]
    pos = deg > 0
    dis = jnp.where(pos, lax.rsqrt(jnp.where(pos, deg, 1.0)), 0.0)
    p_ref[...] = dis * x1 * wg_ref[0, 0]
    dis_ref[...] = dis


_combine = pl.pallas_call(
    _combine_body,
    out_shape=[jax.ShapeDtypeStruct((B, NODE_ATOM), jnp.float32),
               jax.ShapeDtypeStruct((B, NODE_ATOM), jnp.float32)],
)


def _head_body(q0_ref, q1_ref, dis_ref, bg_ref, w3_ref, b3_ref, g_ref, be_ref,
               w4_ref, b4_ref, out_ref):
    q = q0_ref[...] + q1_ref[...]
    xd = jnp.maximum(dis_ref[...] * q + bg_ref[0, 0], 0.0)
    h = lax.dot_general(xd, w3_ref[...], (((1,), (1,)), ((), ())),
                        preferred_element_type=jnp.float32)
    h = h + b3_ref[...]
    h = h * np.float32(1.0 / np.sqrt(1.0 + 1e-5)) * g_ref[...] + be_ref[...]
    h = jnp.maximum(h, 0.0)
    o = lax.dot_general(h, w4_ref[...], (((1,), (1,)), ((), ())),
                        preferred_element_type=jnp.float32)
    out_ref[...] = jnp.maximum(o + b4_ref[...], 0.0)


_head = pl.pallas_call(
    _head_body,
    out_shape=jax.ShapeDtypeStruct((B, 128), jnp.float32),
)


def kernel(x, edge_index, edge_attr, batch, Wf, bf, Ws, bs, Wg, bg,
           W3, b3, gamma, beta, W4, b4):
    xf = x[:, 0]
    src = edge_index[0].reshape(ROWS, 128)
    dst = edge_index[1].reshape(ROWS, 128)
    ea = edge_attr[:, 0].reshape(ROWS, 128)
    w8 = jnp.concatenate([Wf[:, 0], bf, Ws[:, 0], bs])
    w8 = jnp.broadcast_to(w8[:, None], (8, LANES))

    macc0, macc1, dacc0, dacc1 = _pass1(xf, src, dst, ea, w8)
    p2, dis2 = _combine(x.reshape(B, NODE_ATOM),
                        macc0.reshape(B, NODE_ATOM),
                        macc1.reshape(B, NODE_ATOM),
                        dacc0.reshape(B, NODE_ATOM),
                        dacc1.reshape(B, NODE_ATOM),
                        Wg.reshape(1, 1))
    q0, q1 = _pass2(p2.reshape(N), src, dst, ea)
    out = _head(q0.reshape(B, NODE_ATOM), q1.reshape(B, NODE_ATOM),
                dis2, bg.reshape(1, 1),
                W3, b3.reshape(1, -1), gamma.reshape(1, -1),
                beta.reshape(1, -1), W4, b4.reshape(1, -1))
    return out
